# baseline (speedup 1.0000x reference)
import functools

import jax
import jax.numpy as jnp
from jax import lax
from jax.experimental import pallas as pl
from jax.experimental.pallas import tpu as pltpu
from jax.experimental.pallas import tpu_sc as plsc

M = 8192
K = 8192
D = 256
TM = 2048
TK = 2048

NC = 2
NS = 16
NW = NC * NS


def _argmin_body(zsq_ref, wsq_ref, z_ref, w_ref, out_ref):
    zm2 = z_ref[...] * -2.0
    zsq = zsq_ref[...]
    av = jnp.full((TM, 128), jnp.inf, jnp.float32)
    ag = jnp.zeros((TM, 128), jnp.int32)

    for j in range(K // TK):
        dot2 = lax.dot_general(zm2, w_ref[j * TK:(j + 1) * TK, :],
                               (((1,), (1,)), ((), ())),
                               preferred_element_type=jnp.float32)
        for g in range(TK // 128):
            dg = (zsq + wsq_ref[:, j * TK + g * 128:j * TK + (g + 1) * 128]) \
                + dot2[:, g * 128:(g + 1) * 128]
            better = dg < av
            av = jnp.where(better, dg, av)
            ag = jnp.where(better, j * (TK // 128) + g, ag)

    gidx = ag * 128 + lax.broadcasted_iota(jnp.int32, (TM, 128), 1)
    rowmin = jnp.min(av, axis=1, keepdims=True)
    cand = jnp.where(av == rowmin, gidx, K)
    out_ref[...] = jnp.min(cand, axis=1, keepdims=True)


_argmin_call = pl.pallas_call(
    _argmin_body,
    grid=(M // TM,),
    in_specs=[
        pl.BlockSpec((TM, 1), lambda i: (i, 0)),
        pl.BlockSpec((1, K), lambda i: (0, 0)),
        pl.BlockSpec((TM, D), lambda i: (i, 0)),
        pl.BlockSpec((K, D), lambda i: (0, 0)),
    ],
    out_specs=pl.BlockSpec((TM, 1), lambda i: (i, 0)),
    out_shape=jax.ShapeDtypeStruct((M, 1), jnp.int32),
)


@functools.cache
def _make_gather(b):
    b_per_w = b // NW
    mesh = plsc.VectorSubcoreMesh(core_axis_name="c", subcore_axis_name="s")

    @functools.partial(
        pl.kernel,
        mesh=mesh,
        out_type=jax.ShapeDtypeStruct((b, D), jnp.float32),
        scratch_types=[
            pltpu.VMEM((b_per_w,), jnp.int32),
            pltpu.VMEM((b_per_w, D), jnp.float32),
            pltpu.SemaphoreType.DMA,
        ],
    )
    def gather_k(table_hbm, idx_hbm, out_hbm, idx_v, rows_v, sem):
        wid = lax.axis_index("s") * NC + lax.axis_index("c")
        base = wid * b_per_w
        pltpu.sync_copy(idx_hbm.at[pl.ds(base, b_per_w)], idx_v)
        pltpu.async_copy(table_hbm.at[idx_v], rows_v, sem).wait()
        pltpu.sync_copy(rows_v, out_hbm.at[pl.ds(base, b_per_w)])

    return gather_k


def kernel(z, weight):
    B, C, H, W = z.shape
    zt = jnp.transpose(z, (0, 2, 3, 1))
    z_flat = zt.reshape(-1, C)
    zsq = jnp.sum(z_flat ** 2, axis=1, keepdims=True)
    wsq = jnp.sum(weight ** 2, axis=1)[None, :]

    idx = _argmin_call(zsq, wsq, z_flat, weight).reshape(-1)
    z_q = _make_gather(M)(weight, idx)

    z_out = zt.reshape(B, H * W, C)
    return (z_out, z_q.reshape(B, H * W, C), idx.reshape(B, H * W))
